# Initial kernel scaffold; baseline (speedup 1.0000x reference)
import jax
import jax.numpy as jnp
from jax.experimental import pallas as pl


def kernel(encoderout, codebook):
    raise NotImplementedError("write your pallas kernel here")


# speedup vs baseline: 1.4874x; 1.4874x over previous
import functools

import jax
import jax.numpy as jnp
from jax.experimental import pallas as pl
from jax.experimental.pallas import tpu as pltpu

EMBED_DIM = 64
NUM_CODES = 8192
COMMIT_BETA = 0.25


def _vq_body(nk, x_ref, c_ref, acc_ref, minacc_ref):
    i = pl.program_id(0)
    j = pl.program_id(1)
    x = x_ref[...]
    c = c_ref[...]
    dot = jax.lax.dot_general(
        x, c, (((1,), (1,)), ((), ())),
        preferred_element_type=jnp.float32)
    c2 = jnp.sum(c * c, axis=1, keepdims=True)
    part = jnp.min(c2.T - 2.0 * dot, axis=1,
                   keepdims=True)

    @pl.when(j == 0)
    def _initmin():
        minacc_ref[...] = part

    @pl.when(j > 0)
    def _accmin():
        minacc_ref[...] = jnp.minimum(minacc_ref[...], part)

    @pl.when(j == nk - 1)
    def _finish():
        x2 = jnp.sum(x * x, axis=1, keepdims=True)
        tile_sum = jnp.sum(
            jnp.maximum(x2 + minacc_ref[...], 0.0)).reshape(1, 1)

        @pl.when(i == 0)
        def _init():
            acc_ref[...] = tile_sum

        @pl.when(i > 0)
        def _accum():
            acc_ref[...] += tile_sum


@functools.partial(jax.jit, static_argnames=("tm", "tk"))
def _min_dist_sum(flat, codebook, tm=512, tk=1024):
    t = flat.shape[0]
    nk = NUM_CODES // tk
    grid = (t // tm, nk)
    acc = pl.pallas_call(
        functools.partial(_vq_body, nk),
        grid=grid,
        in_specs=[
            pl.BlockSpec((tm, EMBED_DIM), lambda i, j: (i, 0)),
            pl.BlockSpec((tk, EMBED_DIM), lambda i, j: (j, 0)),
        ],
        out_specs=pl.BlockSpec((1, 1), lambda i, j: (0, 0)),
        out_shape=jax.ShapeDtypeStruct((1, 1), jnp.float32),
        scratch_shapes=[pltpu.VMEM((tm, 1), jnp.float32)],
        compiler_params=pltpu.CompilerParams(
            dimension_semantics=("arbitrary", "arbitrary")),
    )(flat, codebook)
    return acc[0, 0]


def kernel(encoderout, codebook):
    x = jnp.transpose(encoderout, (0, 2, 3, 1))
    flat = x.reshape(-1, EMBED_DIM)
    total = _min_dist_sum(flat, codebook)
    mean_sq = total / jnp.float32(flat.size)
    codebook_loss = mean_sq
    commitment_loss = mean_sq
    loss = codebook_loss + COMMIT_BETA * commitment_loss
    return (encoderout, loss, codebook_loss, commitment_loss)


# speedup vs baseline: 2.7282x; 1.8343x over previous
import functools

import jax
import jax.numpy as jnp
from jax.experimental import pallas as pl
from jax.experimental.pallas import tpu as pltpu

EMBED_DIM = 64
NUM_CODES = 8192
COMMIT_BETA = 0.25
AUG = 128


def _vq_body(nk, tk, tm, x_ref, c_ref, acc_ref, ca_ref, xa_ref, minacc_ref):
    i = pl.program_id(0)
    j = pl.program_id(1)

    @pl.when(i == 0)
    def _prep_c():
        cf = c_ref[...]
        c2 = jnp.sum(cf * cf, axis=1, keepdims=True)
        ca = jnp.concatenate(
            [-2.0 * cf, c2, jnp.zeros((tk, AUG - EMBED_DIM - 1), jnp.float32)],
            axis=1)
        ca_ref[pl.ds(j * tk, tk), :] = ca.astype(jnp.bfloat16)

    @pl.when(j == 0)
    def _prep_x():
        xf = x_ref[...]
        xa = jnp.concatenate(
            [xf, jnp.ones((tm, 1), jnp.float32),
             jnp.zeros((tm, AUG - EMBED_DIM - 1), jnp.float32)],
            axis=1)
        xa_ref[...] = xa.astype(jnp.bfloat16)

    dot = jax.lax.dot_general(
        xa_ref[...], ca_ref[pl.ds(j * tk, tk), :],
        (((1,), (1,)), ((), ())),
        preferred_element_type=jnp.float32)
    part = dot[:, 0:128]
    for s in range(128, tk, 128):
        part = jnp.minimum(part, dot[:, s:s + 128])

    @pl.when(j == 0)
    def _initmin():
        minacc_ref[...] = part

    @pl.when(j > 0)
    def _accmin():
        minacc_ref[...] = jnp.minimum(minacc_ref[...], part)

    @pl.when(j == nk - 1)
    def _finish():
        x = x_ref[...]
        x2 = jnp.sum(x * x, axis=1, keepdims=True)
        rowmin = jnp.min(minacc_ref[...], axis=1, keepdims=True)
        tile_sum = jnp.sum(
            jnp.maximum(x2 + rowmin, 0.0)).reshape(1, 1)

        @pl.when(i == 0)
        def _init():
            acc_ref[...] = tile_sum

        @pl.when(i > 0)
        def _accum():
            acc_ref[...] += tile_sum


@functools.partial(jax.jit, static_argnames=("tm", "tk"))
def _min_dist_sum(flat, codebook, tm=512, tk=2048):
    t = flat.shape[0]
    nk = NUM_CODES // tk
    grid = (t // tm, nk)
    acc = pl.pallas_call(
        functools.partial(_vq_body, nk, tk, tm),
        grid=grid,
        in_specs=[
            pl.BlockSpec((tm, EMBED_DIM), lambda i, j: (i, 0)),
            pl.BlockSpec((tk, EMBED_DIM),
                         lambda i, j: (jnp.where(i == 0, j, 0), 0)),
        ],
        out_specs=pl.BlockSpec((1, 1), lambda i, j: (0, 0)),
        out_shape=jax.ShapeDtypeStruct((1, 1), jnp.float32),
        scratch_shapes=[
            pltpu.VMEM((NUM_CODES, AUG), jnp.bfloat16),
            pltpu.VMEM((tm, AUG), jnp.bfloat16),
            pltpu.VMEM((tm, 128), jnp.float32),
        ],
        compiler_params=pltpu.CompilerParams(
            dimension_semantics=("arbitrary", "arbitrary")),
    )(flat, codebook)
    return acc[0, 0]


def kernel(encoderout, codebook):
    x = jnp.transpose(encoderout, (0, 2, 3, 1))
    flat = x.reshape(-1, EMBED_DIM)
    total = _min_dist_sum(flat, codebook)
    mean_sq = total / jnp.float32(flat.size)
    codebook_loss = mean_sq
    commitment_loss = mean_sq
    loss = codebook_loss + COMMIT_BETA * commitment_loss
    return (encoderout, loss, codebook_loss, commitment_loss)
